# baseline (speedup 1.0000x reference)
import jax
import jax.numpy as jnp
from jax import lax
from jax.experimental import pallas as pl
from jax.experimental.pallas import tpu as pltpu
from jax.experimental.pallas import tpu_sc as plsc

N = 10000
E = 320000
D = 128
H = 128
C = 64

NC = 2
NS = 16
NW = NC * NS
CK = 128
NCHUNK = 80
EPTP = NCHUNK * CK
EPAD = NW * EPTP
NPAD = 10240
NPT = NPAD // NS
LANES = 16


def _sc_agg_body(h_hbm, src_hbm, dst_hbm, sums_hbm,
                 srcb, dstb, rows_v, acc_sh,
                 gs0, gs1, ss0, ss1, ds0, ds1):
    c = lax.axis_index("c")
    s = lax.axis_index("s")
    wid = c * NS + s

    gsem = (gs0, gs1)
    ssem = (ss0, ss1)
    dsem = (ds0, ds1)

    def zero_rows(k, _):
        r = k // (D // LANES)
        col = (k % (D // LANES)) * LANES
        rows_v[0, r, pl.ds(col, LANES)] = jnp.zeros((LANES,), jnp.float32)
        return _
    lax.fori_loop(0, CK * (D // LANES), zero_rows, None)
    for k in range(NPT // CK):
        pltpu.sync_copy(rows_v.at[0], acc_sh.at[pl.ds(s * NPT + k * CK, CK), :])

    plsc.subcore_barrier()

    def issue_ids(j, b):
        pltpu.async_copy(src_hbm.at[wid, j], srcb.at[b], ssem[b])
        pltpu.async_copy(dst_hbm.at[wid, j], dstb.at[b], dsem[b])

    def wait_ids(j, b):
        pltpu.make_async_copy(src_hbm.at[wid, j], srcb.at[b], ssem[b]).wait()
        pltpu.make_async_copy(dst_hbm.at[wid, j], dstb.at[b], dsem[b]).wait()

    def issue_gather(j, b):
        pltpu.async_copy(h_hbm.at[srcb.at[b]], rows_v.at[b], gsem[b])

    def wait_gather(j, b):
        pltpu.make_async_copy(h_hbm.at[srcb.at[b]], rows_v.at[b],
                              gsem[b]).wait()

    issue_ids(0, 0)
    issue_ids(1, 1)
    wait_ids(0, 0)
    issue_gather(0, 0)

    def pair(g, _):
        for b in range(2):
            j = 2 * g + b
            nb = 1 - b
            wait_gather(j, b)
            pltpu.sync_copy(rows_v.at[b], acc_sh.at[dstb.at[b]], add=True)
            issue_ids(j + 2, b)
            wait_ids(j + 1, nb)
            issue_gather(j + 1, nb)
        return _
    lax.fori_loop(0, NCHUNK // 2 - 1, pair, None)

    wait_gather(NCHUNK - 2, 0)
    pltpu.sync_copy(rows_v.at[0], acc_sh.at[dstb.at[0]], add=True)
    wait_ids(NCHUNK - 1, 1)
    issue_gather(NCHUNK - 1, 1)
    wait_gather(NCHUNK - 1, 1)
    pltpu.sync_copy(rows_v.at[1], acc_sh.at[dstb.at[1]], add=True)

    plsc.subcore_barrier()

    pltpu.sync_copy(acc_sh.at[pl.ds(s * NPT, NPT), :],
                    sums_hbm.at[c, pl.ds(s * NPT, NPT), :])


def _sc_cnt_body(dst_hbm, cnt_hbm, dst_v, ones_v, cnt_sh, gsem):
    c = lax.axis_index("c")
    s = lax.axis_index("s")
    wid = c * NS + s

    def store_const(val, k, _):
        r = k // (D // LANES)
        col = (k % (D // LANES)) * LANES
        ones_v[r, pl.ds(col, LANES)] = jnp.full((LANES,), val, jnp.float32)
        return _

    lax.fori_loop(0, CK * (D // LANES),
                  lambda k, _: store_const(0.0, k, _), None)
    for k in range(NPT // CK):
        pltpu.sync_copy(ones_v, cnt_sh.at[pl.ds(s * NPT + k * CK, CK), :])

    lax.fori_loop(0, CK * (D // LANES),
                  lambda k, _: store_const(1.0, k, _), None)

    pltpu.sync_copy(dst_hbm.at[wid], dst_v)

    plsc.subcore_barrier()

    def step(j, _):
        pltpu.sync_copy(ones_v, cnt_sh.at[dst_v.at[j]], add=True)
        return _
    lax.fori_loop(0, NCHUNK, step, None)

    plsc.subcore_barrier()

    pltpu.sync_copy(cnt_sh.at[pl.ds(s * NPT, NPT), :],
                    cnt_hbm.at[c, pl.ds(s * NPT, NPT), :])


_sc_mesh = plsc.VectorSubcoreMesh(core_axis_name="c", subcore_axis_name="s")

_sc_agg = pl.kernel(
    _sc_agg_body,
    out_type=(jax.ShapeDtypeStruct((NC, NPAD, D), jnp.float32),),
    mesh=_sc_mesh,
    scratch_types=[
        pltpu.VMEM((2, CK), jnp.int32),
        pltpu.VMEM((2, CK), jnp.int32),
        pltpu.VMEM((2, CK, D), jnp.float32),
        pltpu.VMEM_SHARED((NPAD, D), jnp.float32),
        pltpu.SemaphoreType.DMA,
        pltpu.SemaphoreType.DMA,
        pltpu.SemaphoreType.DMA,
        pltpu.SemaphoreType.DMA,
        pltpu.SemaphoreType.DMA,
        pltpu.SemaphoreType.DMA,
    ],
)

_sc_cnt = pl.kernel(
    _sc_cnt_body,
    out_type=(jax.ShapeDtypeStruct((NC, NPAD, D), jnp.float32),),
    mesh=_sc_mesh,
    scratch_types=[
        pltpu.VMEM((NCHUNK, CK), jnp.int32),
        pltpu.VMEM((CK, D), jnp.float32),
        pltpu.VMEM_SHARED((NPAD, D), jnp.float32),
        pltpu.SemaphoreType.DMA,
    ],
)

RB = 1000


def _tc_layer1_body(sums_ref, cnt_ref, x_ref, wl_ref, bl_ref, wr_ref, o_ref):
    ssum = sums_ref[0] + sums_ref[1]
    cnt = cnt_ref[0] + cnt_ref[1]
    mean = ssum / jnp.maximum(cnt, 1.0)
    h = (jnp.dot(mean, wl_ref[...], preferred_element_type=jnp.float32,
                 precision=lax.Precision.HIGHEST)
         + jnp.dot(x_ref[...], wr_ref[...], preferred_element_type=jnp.float32,
                   precision=lax.Precision.HIGHEST)
         + bl_ref[...])
    o_ref[...] = jnp.maximum(h, 0.0)


def _tc_layer2_body(sums_ref, cnt_ref, h_ref, wl_ref, bl_ref, wr_ref,
                    wo_ref, bo_ref, o_ref):
    ssum = sums_ref[0] + sums_ref[1]
    cnt = cnt_ref[0] + cnt_ref[1]
    mean = ssum / jnp.maximum(cnt, 1.0)
    h = (jnp.dot(mean, wl_ref[...], preferred_element_type=jnp.float32,
                 precision=lax.Precision.HIGHEST)
         + jnp.dot(h_ref[...], wr_ref[...], preferred_element_type=jnp.float32,
                   precision=lax.Precision.HIGHEST)
         + bl_ref[...])
    h = jnp.maximum(h, 0.0)
    o_ref[...] = (jnp.dot(h, wo_ref[...], preferred_element_type=jnp.float32,
                          precision=lax.Precision.HIGHEST)
                  + bo_ref[...])


def _tc_layer1(sums, cnt, x, Wl, bl, Wr):
    grid = (N // RB,)
    return pl.pallas_call(
        _tc_layer1_body,
        grid=grid,
        in_specs=[
            pl.BlockSpec((NC, RB, D), lambda i: (0, i, 0)),
            pl.BlockSpec((NC, RB, D), lambda i: (0, i, 0)),
            pl.BlockSpec((RB, D), lambda i: (i, 0)),
            pl.BlockSpec((D, H), lambda i: (0, 0)),
            pl.BlockSpec((1, H), lambda i: (0, 0)),
            pl.BlockSpec((D, H), lambda i: (0, 0)),
        ],
        out_specs=pl.BlockSpec((RB, H), lambda i: (i, 0)),
        out_shape=jax.ShapeDtypeStruct((N, H), jnp.float32),
    )(sums, cnt, x, Wl, bl, Wr)


def _tc_layer2(sums, cnt, h, Wl, bl, Wr, Wout, bout):
    grid = (N // RB,)
    return pl.pallas_call(
        _tc_layer2_body,
        grid=grid,
        in_specs=[
            pl.BlockSpec((NC, RB, H), lambda i: (0, i, 0)),
            pl.BlockSpec((NC, RB, D), lambda i: (0, i, 0)),
            pl.BlockSpec((RB, H), lambda i: (i, 0)),
            pl.BlockSpec((H, H), lambda i: (0, 0)),
            pl.BlockSpec((1, H), lambda i: (0, 0)),
            pl.BlockSpec((H, H), lambda i: (0, 0)),
            pl.BlockSpec((H, C), lambda i: (0, 0)),
            pl.BlockSpec((1, C), lambda i: (0, 0)),
        ],
        out_specs=pl.BlockSpec((RB, C), lambda i: (i, 0)),
        out_shape=jax.ShapeDtypeStruct((N, C), jnp.float32),
    )(sums, cnt, h, Wl, bl, Wr, Wout, bout)


def kernel(x, edge_index, Wl1, bl1, Wr1, Wl2, bl2, Wr2, Wout, bout):
    npad_e = EPAD - E
    src_p = jnp.concatenate(
        [edge_index[0], jnp.zeros((npad_e,), jnp.int32)])
    dst_p = jnp.concatenate(
        [edge_index[1],
         N + (jnp.arange(npad_e, dtype=jnp.int32) % (NPAD - N))])
    src3 = src_p.reshape(NW, NCHUNK, CK)
    dst3 = dst_p.reshape(NW, NCHUNK, CK)

    (cnt,) = _sc_cnt(dst3)
    (sums1,) = _sc_agg(x, src3, dst3)
    h1 = _tc_layer1(sums1, cnt, x, Wl1, bl1.reshape(1, H), Wr1)
    (sums2,) = _sc_agg(h1, src3, dst3)
    out = _tc_layer2(sums2, cnt, h1, Wl2, bl2.reshape(1, H), Wr2,
                     Wout, bout.reshape(1, C))
    return out
